# baseline (device time: 104568 ns/iter reference)
import jax
import jax.numpy as jnp
from jax import lax
from jax.experimental import pallas as pl
from jax.experimental.pallas import tpu as pltpu

N_DEV = 32
R_HOPS = 16
L_HOPS = 15
N_SEG = 2

def _ham_cycle_coords():
    path = []
    for z in range(4):
        ys = range(4) if z % 2 == 0 else range(3, -1, -1)
        path += [(0, y, z) for y in ys]
    for z in range(3, -1, -1):
        ys = range(4) if z % 2 == 1 else range(3, -1, -1)
        path += [(1, y, z) for y in ys]
    return path


_ROW_OFF = {(0, 0): 0, (1, 0): 1, (1, 1): 2, (0, 1): 3,
            (0, 2): 4, (1, 2): 5, (1, 3): 6, (0, 3): 7}


def _mesh_idx(c):
    x, y, z = c
    return z * 8 + _ROW_OFF[(x, y)]


RING = [_mesh_idx(c) for c in _ham_cycle_coords()]
INV = [0] * N_DEV
for _pos, _m in enumerate(RING):
    INV[_m] = _pos


def kernel(x, w_mat):
    m_per, k = x.shape
    _, n_per = w_mat.shape
    m_seg = m_per // N_SEG

    my = lax.axis_index("i")
    ring = jnp.array(RING, jnp.int32)
    inv = jnp.array(INV, jnp.int32)
    p = inv[my]
    left = ring[(p - 1) % N_DEV]
    right = ring[(p + 1) % N_DEV]
    meta = jnp.concatenate([
        left[None].astype(jnp.int32),
        right[None].astype(jnp.int32),
        my[None].astype(jnp.int32),
        ring[(p - jnp.arange(1, R_HOPS + 1)) % N_DEV],
        ring[(p + jnp.arange(1, L_HOPS + 1)) % N_DEV],
    ])

    def body(x_ref, w_ref, meta_ref, out_ref, comm_ref, send_sems, recv_sems):
        lft = meta_ref[0]
        rgt = meta_ref[1]

        barrier = pltpu.get_barrier_semaphore()
        for nbr in (lft, rgt):
            pl.semaphore_signal(
                barrier, inc=1,
                device_id=(nbr,), device_id_type=pl.DeviceIdType.MESH,
            )
        pl.semaphore_wait(barrier, 2)

        def hop(step, direction, seg):
            src_slot, src_dir = (0, 0) if step == 0 else (step, direction)
            rows = pl.ds(seg * m_seg, m_seg)
            return pltpu.make_async_remote_copy(
                src_ref=comm_ref.at[src_slot, src_dir, rows],
                dst_ref=comm_ref.at[step + 1, direction, rows],
                send_sem=send_sems.at[step + 1, direction, seg],
                recv_sem=recv_sems.at[step + 1, direction, seg],
                device_id=(rgt if direction == 0 else lft,),
                device_id_type=pl.DeviceIdType.MESH,
            )

        def silu_store(y, origin):
            y = y * jax.nn.sigmoid(y)
            out_ref[pl.ds(origin * m_per, m_per), :] = y

        comm_ref[0, 0, :, :] = x_ref[:, :]
        for seg in range(N_SEG):
            hop(0, 0, seg).start()
            hop(0, 1, seg).start()

        y0 = jnp.dot(x_ref[:, :], w_ref[:, :], preferred_element_type=jnp.float32)
        silu_store(y0, meta_ref[2])

        for step in range(1, R_HOPS + 1):
            has_l = step <= L_HOPS
            for seg in range(N_SEG):
                hop(step - 1, 0, seg).wait_recv()
                if step < R_HOPS:
                    hop(step, 0, seg).start()
                if has_l:
                    hop(step - 1, 1, seg).wait_recv()
                    if step < L_HOPS:
                        hop(step, 1, seg).start()
            if step % 2 == 0:
                blk = jnp.reshape(
                    comm_ref[pl.ds(step - 1, 2), :, :, :], (4 * m_per, k)
                )
                y = jnp.dot(blk, w_ref[:, :], preferred_element_type=jnp.float32)
                for b, (s, d) in enumerate(
                    [(step - 1, 0), (step - 1, 1), (step, 0), (step, 1)]
                ):
                    if d == 1 and s > L_HOPS:
                        continue
                    origin = meta_ref[2 + s] if d == 0 else meta_ref[2 + R_HOPS + s]
                    silu_store(y[b * m_per:(b + 1) * m_per, :], origin)

        for step in range(R_HOPS):
            for seg in range(N_SEG):
                hop(step, 0, seg).wait_send()
                if step < L_HOPS:
                    hop(step, 1, seg).wait_send()

    return pl.pallas_call(
        body,
        out_shape=jax.ShapeDtypeStruct((k, n_per), jnp.float32),
        in_specs=[
            pl.BlockSpec(memory_space=pltpu.VMEM),
            pl.BlockSpec(memory_space=pltpu.VMEM),
            pl.BlockSpec(memory_space=pltpu.SMEM),
        ],
        out_specs=pl.BlockSpec(memory_space=pltpu.VMEM),
        scratch_shapes=[
            pltpu.VMEM((R_HOPS + 1, 2, m_per, k), jnp.float32),
            pltpu.SemaphoreType.DMA((R_HOPS + 1, 2, N_SEG)),
            pltpu.SemaphoreType.DMA((R_HOPS + 1, 2, N_SEG)),
        ],
        compiler_params=pltpu.CompilerParams(collective_id=0),
    )(x, w_mat, meta)


# device time: 65711 ns/iter; 1.5913x vs baseline; 1.5913x over previous
import jax
import jax.numpy as jnp
from jax import lax
from jax.experimental import pallas as pl
from jax.experimental.pallas import tpu as pltpu

N_DEV = 32
R_HOPS = 16
L_HOPS = 15
N_SEG = 2

def _ham_cycle_coords():
    path = []
    for z in range(4):
        ys = range(4) if z % 2 == 0 else range(3, -1, -1)
        path += [(0, y, z) for y in ys]
    for z in range(3, -1, -1):
        ys = range(4) if z % 2 == 1 else range(3, -1, -1)
        path += [(1, y, z) for y in ys]
    return path


_ROW_OFF = {(0, 0): 0, (1, 0): 1, (1, 1): 2, (0, 1): 3,
            (0, 2): 4, (1, 2): 5, (1, 3): 6, (0, 3): 7}


def _mesh_idx(c):
    x, y, z = c
    return z * 8 + _ROW_OFF[(x, y)]


RING = [_mesh_idx(c) for c in _ham_cycle_coords()]
INV = [0] * N_DEV
for _pos, _m in enumerate(RING):
    INV[_m] = _pos


def kernel(x, w_mat):
    m_per, k = x.shape
    _, n_per = w_mat.shape
    m_seg = m_per // N_SEG

    my = lax.axis_index("i")
    ring = jnp.array(RING, jnp.int32)
    inv = jnp.array(INV, jnp.int32)
    p = inv[my]
    left = ring[(p - 1) % N_DEV]
    right = ring[(p + 1) % N_DEV]
    meta = jnp.concatenate([
        left[None].astype(jnp.int32),
        right[None].astype(jnp.int32),
        my[None].astype(jnp.int32),
        ring[(p - jnp.arange(1, R_HOPS + 1)) % N_DEV],
        ring[(p + jnp.arange(1, L_HOPS + 1)) % N_DEV],
    ])

    def body(x_ref, w_ref, meta_ref, out_ref, comm_ref, wbf_ref, send_sems, recv_sems):
        lft = meta_ref[0]
        rgt = meta_ref[1]

        barrier = pltpu.get_barrier_semaphore()
        for nbr in (lft, rgt):
            pl.semaphore_signal(
                barrier, inc=1,
                device_id=(nbr,), device_id_type=pl.DeviceIdType.MESH,
            )
        pl.semaphore_wait(barrier, 2)

        def hop(step, direction, seg):
            src_slot, src_dir = (0, 0) if step == 0 else (step, direction)
            rows = pl.ds(seg * m_seg, m_seg)
            return pltpu.make_async_remote_copy(
                src_ref=comm_ref.at[src_slot, src_dir, rows],
                dst_ref=comm_ref.at[step + 1, direction, rows],
                send_sem=send_sems.at[step + 1, direction, seg],
                recv_sem=recv_sems.at[step + 1, direction, seg],
                device_id=(rgt if direction == 0 else lft,),
                device_id_type=pl.DeviceIdType.MESH,
            )

        def silu_store(y, origin):
            y = y * jax.nn.sigmoid(y)
            out_ref[pl.ds(origin * m_per, m_per), :] = y

        comm_ref[0, 0, :, :] = x_ref[:, :].astype(jnp.bfloat16)
        for seg in range(N_SEG):
            hop(0, 0, seg).start()
            hop(0, 1, seg).start()

        wbf_ref[:, :] = w_ref[:, :].astype(jnp.bfloat16)
        y0 = jnp.dot(x_ref[:, :], w_ref[:, :], preferred_element_type=jnp.float32)
        silu_store(y0, meta_ref[2])

        for step in range(1, R_HOPS + 1):
            has_l = step <= L_HOPS
            for seg in range(N_SEG):
                hop(step - 1, 0, seg).wait_recv()
                if step < R_HOPS:
                    hop(step, 0, seg).start()
                if has_l:
                    hop(step - 1, 1, seg).wait_recv()
                    if step < L_HOPS:
                        hop(step, 1, seg).start()
            if step % 2 == 0:
                blk = jnp.reshape(
                    comm_ref[pl.ds(step - 1, 2), :, :, :], (4 * m_per, k)
                )
                y = jnp.dot(blk, wbf_ref[:, :], preferred_element_type=jnp.float32)
                for b, (s, d) in enumerate(
                    [(step - 1, 0), (step - 1, 1), (step, 0), (step, 1)]
                ):
                    if d == 1 and s > L_HOPS:
                        continue
                    origin = meta_ref[2 + s] if d == 0 else meta_ref[2 + R_HOPS + s]
                    silu_store(y[b * m_per:(b + 1) * m_per, :], origin)

        for step in range(R_HOPS):
            for seg in range(N_SEG):
                hop(step, 0, seg).wait_send()
                if step < L_HOPS:
                    hop(step, 1, seg).wait_send()

    return pl.pallas_call(
        body,
        out_shape=jax.ShapeDtypeStruct((k, n_per), jnp.float32),
        in_specs=[
            pl.BlockSpec(memory_space=pltpu.VMEM),
            pl.BlockSpec(memory_space=pltpu.VMEM),
            pl.BlockSpec(memory_space=pltpu.SMEM),
        ],
        out_specs=pl.BlockSpec(memory_space=pltpu.VMEM),
        scratch_shapes=[
            pltpu.VMEM((R_HOPS + 1, 2, m_per, k), jnp.bfloat16),
            pltpu.VMEM((k, n_per), jnp.bfloat16),
            pltpu.SemaphoreType.DMA((R_HOPS + 1, 2, N_SEG)),
            pltpu.SemaphoreType.DMA((R_HOPS + 1, 2, N_SEG)),
        ],
        compiler_params=pltpu.CompilerParams(collective_id=0),
    )(x, w_mat, meta)
